# baseline (speedup 1.0000x reference)
import functools

import jax
import jax.numpy as jnp
from jax import lax
from jax.experimental import pallas as pl
from jax.experimental.pallas import tpu as pltpu
from jax.experimental.pallas import tpu_sc as plsc

N_FULL = 10000
N_COARSE = 2500
E = 320000
F = 128
B = 2
T = 2

NS = 16
C = 80
EPT = E // NS
EPB = 800
NBLK = EPT // EPB
CPB = EPB // C
NC_PAD = 2504
CSTRIDE = 152
OSTRIDE = 624


def _scale_rows(rows, wbig, cbase):
    def body(g, carry):
        w16 = wbig[pl.ds(cbase + g * 16, 16)]
        for lane in range(16):
            wb = w16.at[jnp.full((16,), lane, jnp.int32)].get(
                mode="promise_in_bounds")
            e = g * 16 + lane
            for fb in range(F // 16):
                sl = pl.ds(fb * 16, 16)
                rows[e, sl] = rows[e, sl] * wb
        return carry
    lax.fori_loop(0, C // 16, body, 0)


def _sc_body(x_hbm, dsrc, ddst, dw, usrc, udst, uw, out_hbm,
             sbig, dbig, wbig, didx, didx1, rows, rows1, coarse_sh, full_sh,
             sem, sem1, ssem, ssem1):
    b = lax.axis_index("c")
    t = lax.axis_index("s")
    xoff = (T * b + (T - 1)) * N_FULL

    z16 = jnp.zeros((16,), jnp.float32)

    def zrow(r, carry):
        for fb in range(F // 16):
            rows[r, pl.ds(fb * 16, 16)] = z16
        return carry
    lax.fori_loop(0, C, zrow, 0)

    pltpu.sync_copy(rows, coarse_sh.at[pl.ds(t * CSTRIDE, C)])
    pltpu.sync_copy(rows.at[pl.ds(0, CSTRIDE - C)],
                    coarse_sh.at[pl.ds(t * CSTRIDE + C, CSTRIDE - C)])

    @pl.when(t == 0)
    def _():
        tail = NC_PAD - NS * CSTRIDE
        pltpu.sync_copy(rows.at[pl.ds(0, tail)],
                        coarse_sh.at[pl.ds(NS * CSTRIDE, tail)])

    for k in range(8):
        pltpu.sync_copy(rows, full_sh.at[pl.ds(t * OSTRIDE + k * C, C)])
    plsc.subcore_barrier()

    ebase0 = t * EPT

    def _pass(src_hbm, dst_hbm, w_hbm, gather_from, acc_sh, idx_off):
        def block(blk, carry):
            eb = ebase0 + blk * EPB
            pltpu.sync_copy(src_hbm.at[pl.ds(eb, EPB)], sbig)
            pltpu.sync_copy(dst_hbm.at[pl.ds(eb, EPB)], dbig)
            pltpu.sync_copy(w_hbm.at[pl.ds(eb, EPB)], wbig)

            def off(j, carry2):
                sl = pl.ds(j * 16, 16)
                sbig[sl] = sbig[sl] + idx_off
                return carry2
            lax.fori_loop(0, EPB // 16, off, 0)

            def start_gather(ck, buf, sm):
                pltpu.async_copy(
                    gather_from.at[sbig.at[pl.ds(ck * C, C)]], buf, sm)

            def wait_gather(ck, buf, sm):
                pltpu.make_async_copy(
                    gather_from.at[sbig.at[pl.ds(ck * C, C)]], buf, sm
                ).wait()

            def scale_and_scatter(ck, buf, di, ssm):
                cb = ck * C
                _scale_rows(buf, wbig, cb)
                for j in range(C // 16):
                    sl = pl.ds(j * 16, 16)
                    di[sl] = dbig[pl.ds(cb + j * 16, 16)]
                pltpu.async_copy(buf, acc_sh.at[di], ssm, add=True)

            def wait_scatter(buf, di, ssm):
                pltpu.make_async_copy(buf, acc_sh.at[di], ssm).wait()

            start_gather(0, rows, sem)

            def pair(p, carry2):
                c0 = p * 2

                @pl.when(p > 0)
                def _():
                    wait_scatter(rows1, didx1, ssem1)
                start_gather(c0 + 1, rows1, sem1)
                wait_gather(c0, rows, sem)
                scale_and_scatter(c0, rows, didx, ssem)
                wait_gather(c0 + 1, rows1, sem1)
                scale_and_scatter(c0 + 1, rows1, didx1, ssem1)

                @pl.when(c0 + 2 < CPB)
                def _():
                    wait_scatter(rows, didx, ssem)
                    start_gather(c0 + 2, rows, sem)
                return carry2
            lax.fori_loop(0, CPB // 2, pair, 0)
            wait_scatter(rows, didx, ssem)
            wait_scatter(rows1, didx1, ssem1)
            return carry
        lax.fori_loop(0, NBLK, block, 0)

    _pass(dsrc, ddst, dw, x_hbm, coarse_sh, xoff)
    plsc.subcore_barrier()

    _pass(usrc, udst, uw, coarse_sh, full_sh, 0)
    plsc.subcore_barrier()

    for k in range(8):
        src = pl.ds(t * OSTRIDE + k * C, C)
        pltpu.sync_copy(full_sh.at[src], rows)
        pltpu.sync_copy(rows,
                        out_hbm.at[pl.ds(b * N_FULL + t * OSTRIDE + k * C, C)])


_sc_call = functools.partial(
    pl.kernel,
    out_type=jax.ShapeDtypeStruct((B * N_FULL, F), jnp.float32),
    mesh=plsc.VectorSubcoreMesh(core_axis_name="c", subcore_axis_name="s"),
    scratch_types=[
        pltpu.VMEM((EPB,), jnp.int32),
        pltpu.VMEM((EPB,), jnp.int32),
        pltpu.VMEM((EPB,), jnp.float32),
        pltpu.VMEM((C,), jnp.int32),
        pltpu.VMEM((C,), jnp.int32),
        pltpu.VMEM((C, F), jnp.float32),
        pltpu.VMEM((C, F), jnp.float32),
        pltpu.VMEM_SHARED((NC_PAD, F), jnp.float32),
        pltpu.VMEM_SHARED((N_FULL, F), jnp.float32),
        pltpu.SemaphoreType.DMA,
        pltpu.SemaphoreType.DMA,
        pltpu.SemaphoreType.DMA,
        pltpu.SemaphoreType.DMA,
    ],
)(_sc_body)


def kernel(x, down_src, down_dst, down_w, up_src, up_dst, up_w):
    bsz, tt, ens, n, f = x.shape
    x_flat = x.reshape(bsz * tt * ens * n, f)
    out = _sc_call(x_flat, down_src, down_dst, down_w, up_src, up_dst, up_w)
    return out.reshape(bsz, ens, n, f)


# speedup vs baseline: 1.2347x; 1.2347x over previous
import functools

import jax
import jax.numpy as jnp
from jax import lax
from jax.experimental import pallas as pl
from jax.experimental.pallas import tpu as pltpu
from jax.experimental.pallas import tpu_sc as plsc

N_FULL = 10000
N_COARSE = 2500
E = 320000
F = 128
B = 2
T = 2

NS = 16
C = 80
EPT = E // NS
EPB = 2000
NBLK = EPT // EPB
CPB = EPB // C
NC_PAD = 2504
CSTRIDE = 152
OSTRIDE = 624


def _scale_rows(rows, wbig, cbase):
    def body(g, carry):
        w16 = wbig[pl.ds(cbase + g * 16, 16)]
        for lane in range(16):
            wb = w16.at[jnp.full((16,), lane, jnp.int32)].get(
                mode="promise_in_bounds")
            e = g * 16 + lane
            for fb in range(F // 16):
                sl = pl.ds(fb * 16, 16)
                rows[e, sl] = rows[e, sl] * wb
        return carry
    lax.fori_loop(0, C // 16, body, 0)


def _sc_body(x_hbm, dsrc, ddst, dw, usrc, udst, uw, out_hbm,
             sbig, dbig, wbig, didx, rows, rows1, coarse_sh, full_sh,
             sem, sem1):
    b = lax.axis_index("c")
    t = lax.axis_index("s")
    xoff = (T * b + (T - 1)) * N_FULL

    z16 = jnp.zeros((16,), jnp.float32)

    def zrow(r, carry):
        for fb in range(F // 16):
            rows[r, pl.ds(fb * 16, 16)] = z16
        return carry
    lax.fori_loop(0, C, zrow, 0)

    pltpu.sync_copy(rows, coarse_sh.at[pl.ds(t * CSTRIDE, C)])
    pltpu.sync_copy(rows.at[pl.ds(0, CSTRIDE - C)],
                    coarse_sh.at[pl.ds(t * CSTRIDE + C, CSTRIDE - C)])

    @pl.when(t == 0)
    def _():
        tail = NC_PAD - NS * CSTRIDE
        pltpu.sync_copy(rows.at[pl.ds(0, tail)],
                        coarse_sh.at[pl.ds(NS * CSTRIDE, tail)])

    for k in range(8):
        pltpu.sync_copy(rows, full_sh.at[pl.ds(t * OSTRIDE + k * C, C)])
    plsc.subcore_barrier()

    ebase0 = t * EPT

    def _pass(src_hbm, dst_hbm, w_hbm, gather_from, acc_sh, idx_off):
        def block(blk, carry):
            eb = ebase0 + blk * EPB
            pltpu.sync_copy(src_hbm.at[pl.ds(eb, EPB)], sbig)
            pltpu.sync_copy(dst_hbm.at[pl.ds(eb, EPB)], dbig)
            pltpu.sync_copy(w_hbm.at[pl.ds(eb, EPB)], wbig)

            def off(j, carry2):
                sl = pl.ds(j * 16, 16)
                sbig[sl] = sbig[sl] + idx_off
                return carry2
            lax.fori_loop(0, EPB // 16, off, 0)

            def start_gather(ck, buf, sm):
                pltpu.async_copy(
                    gather_from.at[sbig.at[pl.ds(ck * C, C)]], buf, sm)

            def process(ck, buf, sm):
                pltpu.make_async_copy(
                    gather_from.at[sbig.at[pl.ds(ck * C, C)]], buf, sm
                ).wait()
                cb = ck * C
                _scale_rows(buf, wbig, cb)
                for j in range(C // 16):
                    sl = pl.ds(j * 16, 16)
                    didx[sl] = dbig[pl.ds(cb + j * 16, 16)]
                pltpu.sync_copy(buf, acc_sh.at[didx], add=True)

            start_gather(0, rows, sem)

            def pair(p, carry2):
                c0 = p * 2
                start_gather(c0 + 1, rows1, sem1)
                process(c0, rows, sem)

                start_gather(c0 + 2, rows, sem)
                process(c0 + 1, rows1, sem1)
                return carry2
            lax.fori_loop(0, CPB // 2, pair, 0)
            process(CPB - 1, rows, sem)
            return carry
        lax.fori_loop(0, NBLK, block, 0)

    _pass(dsrc, ddst, dw, x_hbm, coarse_sh, xoff)
    plsc.subcore_barrier()

    _pass(usrc, udst, uw, coarse_sh, full_sh, 0)
    plsc.subcore_barrier()

    for k in range(8):
        src = pl.ds(t * OSTRIDE + k * C, C)
        pltpu.sync_copy(full_sh.at[src], rows)
        pltpu.sync_copy(rows,
                        out_hbm.at[pl.ds(b * N_FULL + t * OSTRIDE + k * C, C)])


_sc_call = functools.partial(
    pl.kernel,
    out_type=jax.ShapeDtypeStruct((B * N_FULL, F), jnp.float32),
    mesh=plsc.VectorSubcoreMesh(core_axis_name="c", subcore_axis_name="s"),
    scratch_types=[
        pltpu.VMEM((EPB,), jnp.int32),
        pltpu.VMEM((EPB,), jnp.int32),
        pltpu.VMEM((EPB,), jnp.float32),
        pltpu.VMEM((C,), jnp.int32),
        pltpu.VMEM((C, F), jnp.float32),
        pltpu.VMEM((C, F), jnp.float32),
        pltpu.VMEM_SHARED((NC_PAD, F), jnp.float32),
        pltpu.VMEM_SHARED((N_FULL, F), jnp.float32),
        pltpu.SemaphoreType.DMA,
        pltpu.SemaphoreType.DMA,
    ],
)(_sc_body)


def kernel(x, down_src, down_dst, down_w, up_src, up_dst, up_w):
    bsz, tt, ens, n, f = x.shape
    x_flat = x.reshape(bsz * tt * ens * n, f)
    out = _sc_call(x_flat, down_src, down_dst, down_w, up_src, up_dst, up_w)
    return out.reshape(bsz, ens, n, f)
